# baseline (device time: 6485 ns/iter reference)
import jax
import jax.numpy as jnp
from jax import lax
from jax.experimental import pallas as pl
from jax.experimental.pallas import tpu as pltpu

N_CHUNKS = 2


def kernel(x):
    m, n = x.shape
    mc = m // N_CHUNKS

    def body(x_hbm, out_ref, xv_ref, acc_ref, recv_ref,
             copy_sems, send_sems, recv_sems):
        my_x = lax.axis_index("x")
        my_y = lax.axis_index("y")
        nbr = (my_x, 1 - my_y)

        barrier_sem = pltpu.get_barrier_semaphore()
        pl.semaphore_signal(
            barrier_sem, inc=1, device_id=nbr,
            device_id_type=pl.DeviceIdType.MESH,
        )

        copies = []
        for c in range(N_CHUNKS):
            cp = pltpu.make_async_copy(
                x_hbm.at[pl.ds(c * mc, mc), :],
                xv_ref.at[c],
                copy_sems.at[c],
            )
            cp.start()
            copies.append(cp)

        ones = jnp.ones((1, n), jnp.float32)
        rdmas = []
        for c in range(N_CHUNKS):
            copies[c].wait()
            acc_ref[:, pl.ds(c * mc, mc)] = lax.dot_general(
                ones, xv_ref[c],
                (((1,), (1,)), ((), ())),
                preferred_element_type=jnp.float32,
            )
            if c == 0:
                pl.semaphore_wait(barrier_sem, 1)
            rdma = pltpu.make_async_remote_copy(
                src_ref=acc_ref.at[:, pl.ds(c * mc, mc)],
                dst_ref=recv_ref.at[:, pl.ds(c * mc, mc)],
                send_sem=send_sems.at[c],
                recv_sem=recv_sems.at[c],
                device_id=nbr,
                device_id_type=pl.DeviceIdType.MESH,
            )
            rdma.start()
            rdmas.append(rdma)

        for c in range(N_CHUNKS):
            rdmas[c].wait()
            out_ref[:, pl.ds(c * mc, mc)] = (
                acc_ref[:, pl.ds(c * mc, mc)]
                + recv_ref[:, pl.ds(c * mc, mc)]
            )

    out_row = pl.pallas_call(
        body,
        out_shape=jax.ShapeDtypeStruct((1, m), jnp.float32),
        in_specs=[pl.BlockSpec(memory_space=pl.ANY)],
        out_specs=pl.BlockSpec(memory_space=pltpu.VMEM),
        scratch_shapes=[
            pltpu.VMEM((N_CHUNKS, mc, n), jnp.float32),
            pltpu.VMEM((1, m), jnp.float32),
            pltpu.VMEM((1, m), jnp.float32),
            pltpu.SemaphoreType.DMA((N_CHUNKS,)),
            pltpu.SemaphoreType.DMA((N_CHUNKS,)),
            pltpu.SemaphoreType.DMA((N_CHUNKS,)),
        ],
        compiler_params=pltpu.CompilerParams(collective_id=0),
    )(x)
    return out_row.reshape(m, 1)
